# Initial kernel scaffold; baseline (speedup 1.0000x reference)
import jax
import jax.numpy as jnp
from jax.experimental import pallas as pl


def kernel(preds, target):
    raise NotImplementedError("write your pallas kernel here")


# speedup vs baseline: 5.4886x; 5.4886x over previous
import jax
import jax.numpy as jnp
from jax import lax
from jax.experimental import pallas as pl
from jax.experimental.pallas import tpu as pltpu

NCLS = 19
HW = 512 * 512
BLK = 16384
NBLK = HW // BLK


def _fused_body(p_ref, t_ref, o_ref, acc_ref):
    j = pl.program_id(1)

    @pl.when(j == 0)
    def _init():
        acc_ref[...] = jnp.zeros_like(acc_ref)

    x = p_ref[0]
    lbl = t_ref[0, 0]

    cidx = lax.broadcasted_iota(jnp.int32, (NCLS, BLK), 0)
    m = jnp.max(x, axis=0, keepdims=True)
    pred = jnp.min(jnp.where(x == m, cidx, NCLS), axis=0, keepdims=True)

    valid = (lbl >= 0) & (lbl < NCLS)
    onehot_l = ((cidx == lbl) & valid).astype(jnp.float32)
    onehot_p = (cidx == pred).astype(jnp.float32)

    conf = lax.dot_general(onehot_l, onehot_p,
                           (((1,), (1,)), ((), ())),
                           preferred_element_type=jnp.float32)
    acc_ref[...] += conf

    @pl.when(j == NBLK - 1)
    def _finish():
        hist = acc_ref[...]
        r_i = lax.broadcasted_iota(jnp.int32, (NCLS, NCLS), 0)
        c_i = lax.broadcasted_iota(jnp.int32, (NCLS, NCLS), 1)
        diag = jnp.sum(jnp.where(r_i == c_i, hist, 0.0), axis=1)
        row = jnp.sum(hist, axis=1)
        col = jnp.sum(hist, axis=0)
        union = jnp.maximum(row + col - diag, 1.0)
        o_ref[...] = (diag / union)[None, None]


def kernel(preds, target):
    b = preds.shape[0]
    preds_r = preds.reshape(b, NCLS, HW)
    target_r = target.reshape(b, NBLK, 1, BLK)
    out = pl.pallas_call(
        _fused_body,
        grid=(b, NBLK),
        in_specs=[
            pl.BlockSpec((1, NCLS, BLK), lambda i, j: (i, 0, j)),
            pl.BlockSpec((1, 1, 1, BLK), lambda i, j: (i, j, 0, 0)),
        ],
        out_specs=pl.BlockSpec((1, 1, NCLS), lambda i, j: (i, 0, 0)),
        out_shape=jax.ShapeDtypeStruct((b, 1, NCLS), jnp.float32),
        scratch_shapes=[pltpu.VMEM((NCLS, NCLS), jnp.float32)],
        compiler_params=pltpu.CompilerParams(
            dimension_semantics=("parallel", "arbitrary")),
    )(preds_r, target_r)
    return out.reshape(b, NCLS)


# speedup vs baseline: 6.3374x; 1.1547x over previous
import functools

import jax
import jax.numpy as jnp
from jax import lax
from jax.experimental import pallas as pl
from jax.experimental.pallas import tpu as pltpu
from jax.experimental.pallas import tpu_sc as plsc

NCLS = 19
HW = 512 * 512
NCHUNK = 16
CHW = HW // NCHUNK

NBINS = 368
NW = 32
EPW = 8 * HW // NW
WPI = NW // 8


def _argmax_body(p_ref, t_ref, o_ref):
    m = p_ref[0, 0, 0]
    pred = jnp.zeros((128, 128), jnp.int32)
    for c in range(1, NCLS):
        xc = p_ref[0, c, 0]
        gt = xc > m
        m = jnp.where(gt, xc, m)
        pred = jnp.where(gt, c, pred)
    lbl = t_ref[0, 0]
    valid = (lbl >= 0) & (lbl < NCLS)
    o_ref[0, 0] = jnp.where(valid, lbl * NCLS + pred, NCLS * NCLS)


def _argmax_idx(preds, target):
    b = preds.shape[0]
    preds_r = preds.reshape(b, NCLS, NCHUNK, 128, 128)
    target_r = target.reshape(b, NCHUNK, 128, 128)
    return pl.pallas_call(
        _argmax_body,
        grid=(b, NCHUNK),
        in_specs=[
            pl.BlockSpec((1, NCLS, 1, 128, 128), lambda i, j: (i, 0, j, 0, 0)),
            pl.BlockSpec((1, 1, 128, 128), lambda i, j: (i, j, 0, 0)),
        ],
        out_specs=pl.BlockSpec((1, 1, 128, 128), lambda i, j: (i, j, 0, 0)),
        out_shape=jax.ShapeDtypeStruct((b, NCHUNK, 128, 128), jnp.int32),
        compiler_params=pltpu.CompilerParams(
            dimension_semantics=("parallel", "parallel")),
    )(preds_r, target_r)


def _sc_hist_body(idx_hbm, out_hbm, idx_v, hist_v, red_v):
    wid = lax.axis_index("s") * 2 + lax.axis_index("c")
    pltpu.sync_copy(idx_hbm.at[pl.ds(wid * EPW, EPW)], idx_v)

    zeros16 = jnp.zeros((16,), jnp.float32)

    def zbody(i, carry):
        hist_v[pl.ds(i * 16, 16)] = zeros16
        return carry

    lax.fori_loop(0, NBINS, zbody, 0)

    lane_base = lax.iota(jnp.int32, 16) * NBINS
    ones16 = jnp.ones((16,), jnp.float32)

    def sbody(i, carry):
        v = idx_v[pl.ds(i * 16, 16)]
        plsc.addupdate_scatter(hist_v, [lane_base + v], ones16)
        return carry

    lax.fori_loop(0, EPW // 16, sbody, 0)

    def rbody(j, carry):
        acc = hist_v[pl.ds(j * 16, 16)]
        for k in range(1, 16):
            acc = acc + hist_v[pl.ds(k * NBINS + j * 16, 16)]
        red_v[pl.ds(j * 16, 16)] = acc
        return carry

    lax.fori_loop(0, NBINS // 16, rbody, 0)
    pltpu.sync_copy(red_v, out_hbm.at[wid])


_sc_hist_cache = {}


def _sc_hist():
    if "k" not in _sc_hist_cache:
        _sc_hist_cache["k"] = functools.partial(
            pl.kernel,
            mesh=plsc.VectorSubcoreMesh(core_axis_name="c",
                                        subcore_axis_name="s"),
            out_type=jax.ShapeDtypeStruct((NW, NBINS), jnp.float32),
            scratch_types=[
                pltpu.VMEM((EPW,), jnp.int32),
                pltpu.VMEM((16 * NBINS,), jnp.float32),
                pltpu.VMEM((NBINS,), jnp.float32),
            ],
            compiler_params=pltpu.CompilerParams(needs_layout_passes=False),
        )(_sc_hist_body)
    return _sc_hist_cache["k"]


def _iou_body(h_ref, o_ref):
    h = jnp.sum(h_ref[0], axis=0, keepdims=True)
    i_i = lax.broadcasted_iota(jnp.int32, (NCLS, NBINS), 0)
    j_i = lax.broadcasted_iota(jnp.int32, (NCLS, NBINS), 1)
    valid_bin = j_i < NCLS * NCLS
    row_m = valid_bin & (j_i // NCLS == i_i)
    col_m = valid_bin & (j_i % NCLS == i_i)
    diag_m = valid_bin & (j_i == i_i * (NCLS + 1))
    row = jnp.sum(jnp.where(row_m, h, 0.0), axis=1)
    col = jnp.sum(jnp.where(col_m, h, 0.0), axis=1)
    diag = jnp.sum(jnp.where(diag_m, h, 0.0), axis=1)
    union = jnp.maximum(row + col - diag, 1.0)
    o_ref[...] = (diag / union)[None, None]


def _iou(parts):
    b = parts.shape[0]
    out = pl.pallas_call(
        _iou_body,
        grid=(b,),
        in_specs=[pl.BlockSpec((1, WPI, NBINS), lambda i: (i, 0, 0))],
        out_specs=pl.BlockSpec((1, 1, NCLS), lambda i: (i, 0, 0)),
        out_shape=jax.ShapeDtypeStruct((b, 1, NCLS), jnp.float32),
    )(parts)
    return out.reshape(b, NCLS)


def kernel(preds, target):
    b = preds.shape[0]
    idx = _argmax_idx(preds, target)
    parts = _sc_hist()(idx.reshape(-1))
    return _iou(parts.reshape(b, WPI, NBINS))


# speedup vs baseline: 16.1729x; 2.5520x over previous
import functools

import jax
import jax.numpy as jnp
from jax import lax
from jax.experimental import pallas as pl
from jax.experimental.pallas import tpu as pltpu
from jax.experimental.pallas import tpu_sc as plsc

NCLS = 19
HW = 512 * 512
NCHUNK = 16
CHW = HW // NCHUNK

NBINS = 368
NW = 32
EPW = 8 * HW // NW
WPI = NW // 8


ROWS = 128


def _argmax_body(p_ref, t_ref, o_ref):
    m = p_ref[0, 0]
    pred = jnp.zeros((ROWS, 512), jnp.int32)
    for c in range(1, NCLS):
        xc = p_ref[0, c]
        gt = xc > m
        m = jnp.where(gt, xc, m)
        pred = jnp.where(gt, c, pred)
    lbl = t_ref[0]
    valid = (lbl >= 0) & (lbl < NCLS)
    o_ref[0] = jnp.where(valid, lbl * NCLS + pred, NCLS * NCLS)


def _argmax_idx(preds, target):
    b, _, h, w = preds.shape
    return pl.pallas_call(
        _argmax_body,
        grid=(b, h // ROWS),
        in_specs=[
            pl.BlockSpec((1, NCLS, ROWS, w), lambda i, j: (i, 0, j, 0)),
            pl.BlockSpec((1, ROWS, w), lambda i, j: (i, j, 0)),
        ],
        out_specs=pl.BlockSpec((1, ROWS, w), lambda i, j: (i, j, 0)),
        out_shape=jax.ShapeDtypeStruct((b, h, w), jnp.int32),
        compiler_params=pltpu.CompilerParams(
            dimension_semantics=("parallel", "parallel")),
    )(preds, target)


def _sc_hist_body(idx_hbm, out_hbm, idx_v, hist_v, red_v):
    wid = lax.axis_index("s") * 2 + lax.axis_index("c")
    pltpu.sync_copy(idx_hbm.at[pl.ds(wid * EPW, EPW)], idx_v)

    zeros16 = jnp.zeros((16,), jnp.float32)

    def zbody(i, carry):
        hist_v[pl.ds(i * 16, 16)] = zeros16
        return carry

    lax.fori_loop(0, NBINS, zbody, 0)

    lane_base = lax.iota(jnp.int32, 16) * NBINS
    ones16 = jnp.ones((16,), jnp.float32)

    def sbody(i, carry):
        v = idx_v[pl.ds(i * 16, 16)]
        plsc.addupdate_scatter(hist_v, [lane_base + v], ones16)
        return carry

    lax.fori_loop(0, EPW // 16, sbody, 0)

    def rbody(j, carry):
        acc = hist_v[pl.ds(j * 16, 16)]
        for k in range(1, 16):
            acc = acc + hist_v[pl.ds(k * NBINS + j * 16, 16)]
        red_v[pl.ds(j * 16, 16)] = acc
        return carry

    lax.fori_loop(0, NBINS // 16, rbody, 0)
    pltpu.sync_copy(red_v, out_hbm.at[wid])


_sc_hist_cache = {}


def _sc_hist():
    if "k" not in _sc_hist_cache:
        _sc_hist_cache["k"] = functools.partial(
            pl.kernel,
            mesh=plsc.VectorSubcoreMesh(core_axis_name="c",
                                        subcore_axis_name="s"),
            out_type=jax.ShapeDtypeStruct((NW, NBINS), jnp.float32),
            scratch_types=[
                pltpu.VMEM((EPW,), jnp.int32),
                pltpu.VMEM((16 * NBINS,), jnp.float32),
                pltpu.VMEM((NBINS,), jnp.float32),
            ],
            compiler_params=pltpu.CompilerParams(needs_layout_passes=False),
        )(_sc_hist_body)
    return _sc_hist_cache["k"]


def _iou_body(h_ref, o_ref):
    h = jnp.sum(h_ref[0], axis=0, keepdims=True)
    i_i = lax.broadcasted_iota(jnp.int32, (NCLS, NBINS), 0)
    j_i = lax.broadcasted_iota(jnp.int32, (NCLS, NBINS), 1)
    valid_bin = j_i < NCLS * NCLS
    row_m = valid_bin & (j_i // NCLS == i_i)
    col_m = valid_bin & (j_i % NCLS == i_i)
    diag_m = valid_bin & (j_i == i_i * (NCLS + 1))
    row = jnp.sum(jnp.where(row_m, h, 0.0), axis=1)
    col = jnp.sum(jnp.where(col_m, h, 0.0), axis=1)
    diag = jnp.sum(jnp.where(diag_m, h, 0.0), axis=1)
    union = jnp.maximum(row + col - diag, 1.0)
    o_ref[...] = (diag / union)[None, None]


def _iou(parts):
    b = parts.shape[0]
    out = pl.pallas_call(
        _iou_body,
        grid=(b,),
        in_specs=[pl.BlockSpec((1, WPI, NBINS), lambda i: (i, 0, 0))],
        out_specs=pl.BlockSpec((1, 1, NCLS), lambda i: (i, 0, 0)),
        out_shape=jax.ShapeDtypeStruct((b, 1, NCLS), jnp.float32),
    )(parts)
    return out.reshape(b, NCLS)


def kernel(preds, target):
    b = preds.shape[0]
    idx = _argmax_idx(preds, target)
    parts = _sc_hist()(idx.reshape(-1))
    return _iou(parts.reshape(b, WPI, NBINS))


# speedup vs baseline: 16.8529x; 1.0420x over previous
import functools

import jax
import jax.numpy as jnp
from jax import lax
from jax.experimental import pallas as pl
from jax.experimental.pallas import tpu as pltpu
from jax.experimental.pallas import tpu_sc as plsc

NCLS = 19
HW = 512 * 512
NCHUNK = 16
CHW = HW // NCHUNK

NBINS = 368
NW = 32
EPW = 8 * HW // NW
WPI = NW // 8


ROWS = 128


def _argmax_body(p_ref, t_ref, o_ref):
    m = p_ref[0, 0]
    pred = jnp.zeros((ROWS, 512), jnp.int32)
    for c in range(1, NCLS):
        xc = p_ref[0, c]
        gt = xc > m
        m = jnp.where(gt, xc, m)
        pred = jnp.where(gt, c, pred)
    lbl = t_ref[0]
    valid = (lbl >= 0) & (lbl < NCLS)
    o_ref[0] = jnp.where(valid, lbl * NCLS + pred, NCLS * NCLS)


def _argmax_idx(preds, target):
    b, _, h, w = preds.shape
    return pl.pallas_call(
        _argmax_body,
        grid=(b, h // ROWS),
        in_specs=[
            pl.BlockSpec((1, NCLS, ROWS, w), lambda i, j: (i, 0, j, 0)),
            pl.BlockSpec((1, ROWS, w), lambda i, j: (i, j, 0)),
        ],
        out_specs=pl.BlockSpec((1, ROWS, w), lambda i, j: (i, j, 0)),
        out_shape=jax.ShapeDtypeStruct((b, h, w), jnp.int32),
        compiler_params=pltpu.CompilerParams(
            dimension_semantics=("parallel", "parallel")),
    )(preds, target)


def _sc_hist_body(idx_hbm, out_hbm, idx_v, hist_v, red_v):
    wid = lax.axis_index("s") * 2 + lax.axis_index("c")
    pltpu.sync_copy(idx_hbm.at[pl.ds(wid * EPW, EPW)], idx_v)

    zeros16 = jnp.zeros((16,), jnp.float32)

    def zbody(i, carry):
        hist_v[pl.ds(i * 16, 16)] = zeros16
        return carry

    lax.fori_loop(0, NBINS, zbody, 0, unroll=8)

    lane_base = lax.iota(jnp.int32, 16) * NBINS
    ones16 = jnp.ones((16,), jnp.float32)

    def sbody(i, carry):
        v = idx_v[pl.ds(i * 16, 16)]
        plsc.addupdate_scatter(hist_v, [lane_base + v], ones16)
        return carry

    lax.fori_loop(0, EPW // 16, sbody, 0, unroll=16)

    def rbody(j, carry):
        acc = hist_v[pl.ds(j * 16, 16)]
        for k in range(1, 16):
            acc = acc + hist_v[pl.ds(k * NBINS + j * 16, 16)]
        red_v[pl.ds(j * 16, 16)] = acc
        return carry

    lax.fori_loop(0, NBINS // 16, rbody, 0)
    pltpu.sync_copy(red_v, out_hbm.at[wid])


_sc_hist_cache = {}


def _sc_hist():
    if "k" not in _sc_hist_cache:
        _sc_hist_cache["k"] = functools.partial(
            pl.kernel,
            mesh=plsc.VectorSubcoreMesh(core_axis_name="c",
                                        subcore_axis_name="s"),
            out_type=jax.ShapeDtypeStruct((NW, NBINS), jnp.float32),
            scratch_types=[
                pltpu.VMEM((EPW,), jnp.int32),
                pltpu.VMEM((16 * NBINS,), jnp.float32),
                pltpu.VMEM((NBINS,), jnp.float32),
            ],
            compiler_params=pltpu.CompilerParams(needs_layout_passes=False),
        )(_sc_hist_body)
    return _sc_hist_cache["k"]


def _iou_body(h_ref, o_ref):
    h = jnp.sum(h_ref[0], axis=0, keepdims=True)
    i_i = lax.broadcasted_iota(jnp.int32, (NCLS, NBINS), 0)
    j_i = lax.broadcasted_iota(jnp.int32, (NCLS, NBINS), 1)
    valid_bin = j_i < NCLS * NCLS
    row_m = valid_bin & (j_i // NCLS == i_i)
    col_m = valid_bin & (j_i % NCLS == i_i)
    diag_m = valid_bin & (j_i == i_i * (NCLS + 1))
    row = jnp.sum(jnp.where(row_m, h, 0.0), axis=1)
    col = jnp.sum(jnp.where(col_m, h, 0.0), axis=1)
    diag = jnp.sum(jnp.where(diag_m, h, 0.0), axis=1)
    union = jnp.maximum(row + col - diag, 1.0)
    o_ref[...] = (diag / union)[None, None]


def _iou(parts):
    b = parts.shape[0]
    out = pl.pallas_call(
        _iou_body,
        grid=(b,),
        in_specs=[pl.BlockSpec((1, WPI, NBINS), lambda i: (i, 0, 0))],
        out_specs=pl.BlockSpec((1, 1, NCLS), lambda i: (i, 0, 0)),
        out_shape=jax.ShapeDtypeStruct((b, 1, NCLS), jnp.float32),
    )(parts)
    return out.reshape(b, NCLS)


def kernel(preds, target):
    b = preds.shape[0]
    idx = _argmax_idx(preds, target)
    parts = _sc_hist()(idx.reshape(-1))
    return _iou(parts.reshape(b, WPI, NBINS))


# speedup vs baseline: 24.2261x; 1.4375x over previous
import functools

import jax
import jax.numpy as jnp
from jax import lax
from jax.experimental import pallas as pl
from jax.experimental.pallas import tpu as pltpu
from jax.experimental.pallas import tpu_sc as plsc

NCLS = 19
HW = 512 * 512
NBINS = 368

ROWS = 256


def _argmax_body(p_ref, t_ref, o_ref):
    m = p_ref[0, 0]
    pred = jnp.zeros((ROWS, 512), jnp.int32)
    for c in range(1, NCLS):
        xc = p_ref[0, c]
        gt = xc > m
        m = jnp.where(gt, xc, m)
        pred = jnp.where(gt, c, pred)
    lbl = t_ref[0]
    valid = (lbl >= 0) & (lbl < NCLS)
    idx = jnp.where(valid, lbl * NCLS + pred, NCLS * NCLS)
    o_ref[...] = idx.reshape(ROWS * 4, 128)


def _argmax_idx(preds, target, b0, nb):
    _, _, h, w = preds.shape
    nj = h // ROWS
    return pl.pallas_call(
        _argmax_body,
        grid=(nb, nj),
        in_specs=[
            pl.BlockSpec((1, NCLS, ROWS, w), lambda i, j: (b0 + i, 0, j, 0)),
            pl.BlockSpec((1, ROWS, w), lambda i, j: (b0 + i, j, 0)),
        ],
        out_specs=pl.BlockSpec((ROWS * 4, 128), lambda i, j: (i * nj + j, 0)),
        out_shape=jax.ShapeDtypeStruct((nb * h * w // 128, 128), jnp.int32),
        compiler_params=pltpu.CompilerParams(
            dimension_semantics=("parallel", "parallel")),
    )(preds, target)


def _make_sc_body(nimg):
    epw = nimg * HW // 32
    wpi = 32 // nimg
    ipc = nimg // 2

    def body(idx_hbm, out_hbm, idx_v, hist_v, red_v, part_v, out_v, shared):
        cid = lax.axis_index("c")
        sid = lax.axis_index("s")
        wid = cid * 16 + sid
        rows = epw // 128
        pltpu.sync_copy(idx_hbm.at[pl.ds(wid * rows, rows)], idx_v)

        zeros16 = jnp.zeros((16,), jnp.float32)

        def zbody(i, carry):
            hist_v[pl.ds(i * 16, 16)] = zeros16
            return carry

        lax.fori_loop(0, NBINS, zbody, 0, unroll=8)

        lane_base = lax.iota(jnp.int32, 16) * NBINS
        ones16 = jnp.ones((16,), jnp.float32)

        def sbody(i, carry):
            vs = [idx_v[i * 2 + rr, pl.ds(k * 16, 16)]
                  for rr in range(2) for k in range(8)]
            ts = [lane_base + v for v in vs]
            for t in ts:
                plsc.addupdate_scatter(hist_v, [t], ones16)
            return carry

        lax.fori_loop(0, epw // 256, sbody, 0)

        def rbody(j, carry):
            acc = hist_v[pl.ds(j * 16, 16)]
            for k in range(1, 16):
                acc = acc + hist_v[pl.ds(k * NBINS + j * 16, 16)]
            red_v[pl.ds(j * 16, 16)] = acc
            return carry

        lax.fori_loop(0, NBINS // 16, rbody, 0)

        pltpu.sync_copy(red_v, shared.at[pl.ds(sid * NBINS, NBINS)])
        plsc.subcore_barrier()

        @pl.when(sid % wpi == 0)
        def _leader():
            img = cid * ipc + sid // wpi
            pltpu.sync_copy(shared.at[pl.ds(sid * NBINS, wpi * NBINS)],
                            part_v)

            def hbody(j, carry):
                acc = part_v[pl.ds(j * 16, 16)]
                for k in range(1, wpi):
                    acc = acc + part_v[pl.ds(k * NBINS + j * 16, 16)]
                red_v[pl.ds(j * 16, 16)] = acc
                return carry

            lax.fori_loop(0, NBINS // 16, hbody, 0)

            lanes = lax.iota(jnp.int32, 16)
            for c2 in range(2):
                cls = jnp.minimum(lanes + 16 * c2, NCLS - 1)
                row = plsc.load_gather(red_v, [cls * NCLS])
                col = plsc.load_gather(red_v, [cls])
                for j in range(1, NCLS):
                    row = row + plsc.load_gather(red_v, [cls * NCLS + j])
                    col = col + plsc.load_gather(red_v, [j * NCLS + cls])
                diag = plsc.load_gather(red_v, [cls * (NCLS + 1)])
                union = jnp.maximum(row + col - diag, 1.0)
                out_v[pl.ds(16 * c2, 16)] = diag / union
            pltpu.sync_copy(out_v, out_hbm.at[pl.ds(img * 32, 32)])

    return body


_sc_hist_cache = {}


def _sc_hist(nimg):
    if nimg not in _sc_hist_cache:
        epw = nimg * HW // 32
        wpi = 32 // nimg
        _sc_hist_cache[nimg] = functools.partial(
            pl.kernel,
            mesh=plsc.VectorSubcoreMesh(core_axis_name="c",
                                        subcore_axis_name="s"),
            out_type=jax.ShapeDtypeStruct((nimg * 32,), jnp.float32),
            scratch_types=[
                pltpu.VMEM((epw // 128, 128), jnp.int32),
                pltpu.VMEM((16 * NBINS,), jnp.float32),
                pltpu.VMEM((NBINS,), jnp.float32),
                pltpu.VMEM((wpi * NBINS,), jnp.float32),
                pltpu.VMEM((32,), jnp.float32),
                pltpu.VMEM_SHARED((16 * NBINS,), jnp.float32),
            ],
            compiler_params=pltpu.CompilerParams(needs_layout_passes=False),
        )(_make_sc_body(nimg))
    return _sc_hist_cache[nimg]


def kernel(preds, target):
    b = preds.shape[0]
    half = b // 2
    idx0 = _argmax_idx(preds, target, 0, half)
    idx1 = _argmax_idx(preds, target, half, half)
    h0 = _sc_hist(half)(idx0)
    h1 = _sc_hist(half)(idx1)
    out = jnp.concatenate([h0, h1]).reshape(b, 32)
    return out[:, :NCLS]


# speedup vs baseline: 24.2687x; 1.0018x over previous
import functools

import jax
import jax.numpy as jnp
from jax import lax
from jax.experimental import pallas as pl
from jax.experimental.pallas import tpu as pltpu
from jax.experimental.pallas import tpu_sc as plsc

NCLS = 19
HW = 512 * 512
NBINS = 368

ROWS = 256


def _argmax_body(p_ref, t_ref, o_ref):
    m = p_ref[0, 0]
    pred = jnp.zeros((ROWS, 512), jnp.int32)
    for c in range(1, NCLS):
        xc = p_ref[0, c]
        gt = xc > m
        m = jnp.where(gt, xc, m)
        pred = jnp.where(gt, c, pred)
    lbl = t_ref[0]
    valid = (lbl >= 0) & (lbl < NCLS)
    idx = jnp.where(valid, lbl * NCLS + pred, NCLS * NCLS)
    o_ref[...] = idx.reshape(ROWS * 4, 128)


def _argmax_idx(preds, target, b0, nb):
    _, _, h, w = preds.shape
    nj = h // ROWS
    return pl.pallas_call(
        _argmax_body,
        grid=(nb, nj),
        in_specs=[
            pl.BlockSpec((1, NCLS, ROWS, w), lambda i, j: (b0 + i, 0, j, 0)),
            pl.BlockSpec((1, ROWS, w), lambda i, j: (b0 + i, j, 0)),
        ],
        out_specs=pl.BlockSpec((ROWS * 4, 128), lambda i, j: (i * nj + j, 0)),
        out_shape=jax.ShapeDtypeStruct((nb * h * w // 128, 128), jnp.int32),
        compiler_params=pltpu.CompilerParams(
            dimension_semantics=("parallel", "parallel")),
    )(preds, target)


def _make_sc_body(nimg):
    epw = nimg * HW // 32
    wpi = 32 // nimg
    ipc = nimg // 2

    def body(idx_hbm, out_hbm, idx_v, hist_v, red_v, part_v, out_v, shared):
        cid = lax.axis_index("c")
        sid = lax.axis_index("s")
        wid = cid * 16 + sid
        rows = epw // 128
        pltpu.sync_copy(idx_hbm.at[pl.ds(wid * rows, rows)], idx_v)

        zeros16 = jnp.zeros((16,), jnp.float32)

        def zbody(i, carry):
            hist_v[pl.ds(i * 16, 16)] = zeros16
            return carry

        lax.fori_loop(0, NBINS, zbody, 0, unroll=8)

        lane_base = lax.iota(jnp.int32, 16) * NBINS
        ones16 = jnp.ones((16,), jnp.float32)

        def sbody(i, carry):
            vs = [idx_v[i * 2 + rr, pl.ds(k * 16, 16)]
                  for rr in range(2) for k in range(8)]
            ts = [lane_base + v for v in vs]
            for t in ts:
                plsc.addupdate_scatter(hist_v, [t], ones16)
            return carry

        lax.fori_loop(0, epw // 256, sbody, 0)

        def rbody(j, carry):
            acc = hist_v[pl.ds(j * 16, 16)]
            for k in range(1, 16):
                acc = acc + hist_v[pl.ds(k * NBINS + j * 16, 16)]
            red_v[pl.ds(j * 16, 16)] = acc
            return carry

        lax.fori_loop(0, NBINS // 16, rbody, 0)

        pltpu.sync_copy(red_v, shared.at[pl.ds(sid * NBINS, NBINS)])
        plsc.subcore_barrier()

        @pl.when(sid % wpi == 0)
        def _leader():
            img = cid * ipc + sid // wpi
            pltpu.sync_copy(shared.at[pl.ds(sid * NBINS, wpi * NBINS)],
                            part_v)

            def hbody(j, carry):
                acc = part_v[pl.ds(j * 16, 16)]
                for k in range(1, wpi):
                    acc = acc + part_v[pl.ds(k * NBINS + j * 16, 16)]
                red_v[pl.ds(j * 16, 16)] = acc
                return carry

            lax.fori_loop(0, NBINS // 16, hbody, 0)

            lanes = lax.iota(jnp.int32, 16)
            for c2 in range(2):
                cls = jnp.minimum(lanes + 16 * c2, NCLS - 1)
                row = plsc.load_gather(red_v, [cls * NCLS])
                col = plsc.load_gather(red_v, [cls])
                for j in range(1, NCLS):
                    row = row + plsc.load_gather(red_v, [cls * NCLS + j])
                    col = col + plsc.load_gather(red_v, [j * NCLS + cls])
                diag = plsc.load_gather(red_v, [cls * (NCLS + 1)])
                union = jnp.maximum(row + col - diag, 1.0)
                out_v[pl.ds(16 * c2, 16)] = diag / union
            pltpu.sync_copy(out_v, out_hbm.at[pl.ds(img * 32, 32)])

    return body


_sc_hist_cache = {}


def _sc_hist(nimg):
    if nimg not in _sc_hist_cache:
        epw = nimg * HW // 32
        wpi = 32 // nimg
        _sc_hist_cache[nimg] = functools.partial(
            pl.kernel,
            mesh=plsc.VectorSubcoreMesh(core_axis_name="c",
                                        subcore_axis_name="s"),
            out_type=jax.ShapeDtypeStruct((nimg * 32,), jnp.float32),
            scratch_types=[
                pltpu.VMEM((epw // 128, 128), jnp.int32),
                pltpu.VMEM((16 * NBINS,), jnp.float32),
                pltpu.VMEM((NBINS,), jnp.float32),
                pltpu.VMEM((wpi * NBINS,), jnp.float32),
                pltpu.VMEM((32,), jnp.float32),
                pltpu.VMEM_SHARED((16 * NBINS,), jnp.float32),
            ],
            compiler_params=pltpu.CompilerParams(needs_layout_passes=False),
        )(_make_sc_body(nimg))
    return _sc_hist_cache[nimg]


def kernel(preds, target):
    b = preds.shape[0]
    idx = _argmax_idx(preds, target, 0, b)
    out = _sc_hist(b)(idx)
    return out.reshape(b, 32)[:, :NCLS]


# speedup vs baseline: 24.6291x; 1.0149x over previous
import functools

import jax
import jax.numpy as jnp
from jax import lax
from jax.experimental import pallas as pl
from jax.experimental.pallas import tpu as pltpu
from jax.experimental.pallas import tpu_sc as plsc

NCLS = 19
HW = 512 * 512
NBINS = 368

ROWS = 256


def _argmax_body(p_ref, t_ref, o_ref):
    m = p_ref[0, 0]
    pred = jnp.zeros((ROWS, 512), jnp.int32)
    for c in range(1, NCLS):
        xc = p_ref[0, c]
        gt = xc > m
        m = jnp.where(gt, xc, m)
        pred = jnp.where(gt, c, pred)
    lbl = t_ref[0]
    valid = (lbl >= 0) & (lbl < NCLS)
    idx = jnp.where(valid, lbl * NCLS + pred, NCLS * NCLS)
    o_ref[...] = idx.reshape(ROWS * 4, 128)


def _argmax_idx(preds, target, b0, nb):
    _, _, h, w = preds.shape
    nj = h // ROWS
    return pl.pallas_call(
        _argmax_body,
        grid=(nb, nj),
        in_specs=[
            pl.BlockSpec((1, NCLS, ROWS, w), lambda i, j: (b0 + i, 0, j, 0)),
            pl.BlockSpec((1, ROWS, w), lambda i, j: (b0 + i, j, 0)),
        ],
        out_specs=pl.BlockSpec((ROWS * 4, 128), lambda i, j: (i * nj + j, 0)),
        out_shape=jax.ShapeDtypeStruct((nb * h * w // 128, 128), jnp.int32),
        compiler_params=pltpu.CompilerParams(
            dimension_semantics=("parallel", "parallel")),
    )(preds, target)


def _make_sc_body(nimg):
    epw = nimg * HW // 32
    wpi = 32 // nimg
    ipc = nimg // 2

    nch = 4
    crows = epw // 128 // nch

    def body(idx_hbm, out_hbm, idx_a, idx_b, hist_v, red_v, part_v, out_v,
             shared, sem_a, sem_b):
        cid = lax.axis_index("c")
        sid = lax.axis_index("s")
        wid = cid * 16 + sid
        base = wid * epw // 128
        bufs = [idx_a, idx_b]
        sems = [sem_a, sem_b]

        h = pltpu.async_copy(
            idx_hbm.at[pl.ds(pl.multiple_of(base, 8), crows)], idx_a, sem_a)

        zeros16 = jnp.zeros((16,), jnp.float32)

        def zbody(i, carry):
            hist_v[pl.ds(i * 16, 16)] = zeros16
            return carry

        lax.fori_loop(0, NBINS, zbody, 0, unroll=8)

        lane_base = lax.iota(jnp.int32, 16) * NBINS
        ones16 = jnp.ones((16,), jnp.float32)

        def scatter_chunk(buf):
            def sbody(i, carry):
                vs = [buf[i * 2 + rr, pl.ds(k * 16, 16)]
                      for rr in range(2) for k in range(8)]
                ts = [lane_base + v for v in vs]
                for t in ts:
                    plsc.addupdate_scatter(hist_v, [t], ones16)
                return carry

            lax.fori_loop(0, crows // 2, sbody, 0)

        for ch in range(nch):
            nxt = None
            if ch + 1 < nch:
                nxt = pltpu.async_copy(
                    idx_hbm.at[pl.ds(
                        pl.multiple_of(base + (ch + 1) * crows, 8), crows)],
                    bufs[(ch + 1) % 2], sems[(ch + 1) % 2])
            h.wait()
            scatter_chunk(bufs[ch % 2])
            h = nxt

        def rbody(j, carry):
            acc = hist_v[pl.ds(j * 16, 16)]
            for k in range(1, 16):
                acc = acc + hist_v[pl.ds(k * NBINS + j * 16, 16)]
            red_v[pl.ds(j * 16, 16)] = acc
            return carry

        lax.fori_loop(0, NBINS // 16, rbody, 0)

        pltpu.sync_copy(red_v, shared.at[pl.ds(sid * NBINS, NBINS)])
        plsc.subcore_barrier()

        @pl.when(sid % wpi == 0)
        def _leader():
            img = cid * ipc + sid // wpi
            pltpu.sync_copy(shared.at[pl.ds(sid * NBINS, wpi * NBINS)],
                            part_v)

            def hbody(j, carry):
                acc = part_v[pl.ds(j * 16, 16)]
                for k in range(1, wpi):
                    acc = acc + part_v[pl.ds(k * NBINS + j * 16, 16)]
                red_v[pl.ds(j * 16, 16)] = acc
                return carry

            lax.fori_loop(0, NBINS // 16, hbody, 0)

            lanes = lax.iota(jnp.int32, 16)
            for c2 in range(2):
                cls = jnp.minimum(lanes + 16 * c2, NCLS - 1)
                row = plsc.load_gather(red_v, [cls * NCLS])
                col = plsc.load_gather(red_v, [cls])
                for j in range(1, NCLS):
                    row = row + plsc.load_gather(red_v, [cls * NCLS + j])
                    col = col + plsc.load_gather(red_v, [j * NCLS + cls])
                diag = plsc.load_gather(red_v, [cls * (NCLS + 1)])
                union = jnp.maximum(row + col - diag, 1.0)
                out_v[pl.ds(16 * c2, 16)] = diag / union
            pltpu.sync_copy(out_v, out_hbm.at[pl.ds(img * 32, 32)])

    return body


_sc_hist_cache = {}


def _sc_hist(nimg):
    if nimg not in _sc_hist_cache:
        epw = nimg * HW // 32
        wpi = 32 // nimg
        _sc_hist_cache[nimg] = functools.partial(
            pl.kernel,
            mesh=plsc.VectorSubcoreMesh(core_axis_name="c",
                                        subcore_axis_name="s"),
            out_type=jax.ShapeDtypeStruct((nimg * 32,), jnp.float32),
            scratch_types=[
                pltpu.VMEM((epw // 128 // 4, 128), jnp.int32),
                pltpu.VMEM((epw // 128 // 4, 128), jnp.int32),
                pltpu.VMEM((16 * NBINS,), jnp.float32),
                pltpu.VMEM((NBINS,), jnp.float32),
                pltpu.VMEM((wpi * NBINS,), jnp.float32),
                pltpu.VMEM((32,), jnp.float32),
                pltpu.VMEM_SHARED((16 * NBINS,), jnp.float32),
                pltpu.SemaphoreType.DMA,
                pltpu.SemaphoreType.DMA,
            ],
            compiler_params=pltpu.CompilerParams(needs_layout_passes=False),
        )(_make_sc_body(nimg))
    return _sc_hist_cache[nimg]


def kernel(preds, target):
    b = preds.shape[0]
    idx = _argmax_idx(preds, target, 0, b)
    out = _sc_hist(b)(idx)
    return out.reshape(b, 32)[:, :NCLS]
